# baseline (speedup 1.0000x reference)
import functools

import jax
import jax.numpy as jnp
from jax import lax
from jax.experimental import pallas as pl
from jax.experimental.pallas import tpu as pltpu
from jax.experimental.pallas import tpu_sc as plsc

N_NODES = 100000
D = 128
D_OUT = 128
B = 20000
K = 25

NC = 2
NS = 16
NW = NC * NS
BP = 20480
PER_W = BP // NW
C = 128
NCH = PER_W // C


NBUF = 4
NPAIR = K // 2


def _sc_body(feat, nodesp, neigh_t, xs_out, xsum_out,
             idxv, sidx, planes, selfbuf, acc,
             sem_s, sem_p0, sem_p1, sem_p2, sem_p3):
    wid = lax.axis_index("s") * NC + lax.axis_index("c")
    psems = [sem_p0, sem_p1, sem_p2, sem_p3]

    def chunk(i, carry):
        base = wid * PER_W + i * C
        pltpu.sync_copy(neigh_t.at[:, pl.ds(base, C)], idxv)
        pltpu.sync_copy(nodesp.at[pl.ds(base, C)], sidx)
        cp_self = pltpu.async_copy(feat.at[sidx], selfbuf, sem_s)
        cps = [None] * NBUF
        for b in range(NBUF):
            cps[b] = pltpu.async_copy(feat.at[idxv.at[b]], planes.at[b],
                                      psems[b])
        for kp in range(NPAIR):
            ba = (2 * kp) % NBUF
            bb = (2 * kp + 1) % NBUF
            cps[ba].wait()
            cps[bb].wait()
            first = kp == 0

            @plsc.parallel_loop(0, C, unroll=2)
            def _(c, _ba=ba, _bb=bb, _first=first):
                for j in range(8):
                    sl = pl.ds(j * 16, 16)
                    v = planes[_ba, c, sl] + planes[_bb, c, sl]
                    if _first:
                        acc[c, sl] = v
                    else:
                        plsc.addupdate(acc.at[c, sl], v)

            for k in (2 * kp + NBUF, 2 * kp + NBUF + 1):
                if k < K:
                    cps[k % NBUF] = pltpu.async_copy(
                        feat.at[idxv.at[k]], planes.at[k % NBUF],
                        psems[k % NBUF])
        cps[(K - 1) % NBUF].wait()

        @plsc.parallel_loop(0, C, unroll=2)
        def _(c):
            for j in range(8):
                sl = pl.ds(j * 16, 16)
                plsc.addupdate(acc.at[c, sl], planes[(K - 1) % NBUF, c, sl])

        cp_self.wait()
        pltpu.sync_copy(acc, xsum_out.at[pl.ds(base, C)])
        pltpu.sync_copy(selfbuf, xs_out.at[pl.ds(base, C)])
        return carry

    lax.fori_loop(0, NCH, chunk, 0)


def _sc_gather(feat, nodesp, neigh_t):
    mesh = plsc.VectorSubcoreMesh(core_axis_name="c", subcore_axis_name="s")
    f = pl.kernel(
        _sc_body, mesh=mesh,
        out_type=(jax.ShapeDtypeStruct((BP, D), jnp.float32),
                  jax.ShapeDtypeStruct((BP, D), jnp.float32)),
        scratch_types=[
            pltpu.VMEM((K, C), jnp.int32),
            pltpu.VMEM((C,), jnp.int32),
            pltpu.VMEM((NBUF, C, D), jnp.float32),
            pltpu.VMEM((C, D), jnp.float32),
            pltpu.VMEM((C, D), jnp.float32),
            pltpu.SemaphoreType.DMA,
            pltpu.SemaphoreType.DMA,
            pltpu.SemaphoreType.DMA,
            pltpu.SemaphoreType.DMA,
            pltpu.SemaphoreType.DMA,
        ],
    )
    return f(feat, nodesp, neigh_t)


def _mm_body(xs_ref, xm_ref, ws_ref, wr_ref, o_ref):
    a = lax.dot_general(ws_ref[...], xs_ref[...],
                        (((1,), (1,)), ((), ())),
                        preferred_element_type=jnp.float32)
    b = lax.dot_general(wr_ref[...], xm_ref[...],
                        (((1,), (1,)), ((), ())),
                        preferred_element_type=jnp.float32)
    o_ref[...] = jnp.maximum(0.5 * a + (0.5 / K) * b, 0.0)


def _tc_combine(xs, xm, w_self, w_rel):
    tb = 2560
    grid = BP // tb
    return pl.pallas_call(
        _mm_body,
        grid=(grid,),
        in_specs=[
            pl.BlockSpec((tb, D), lambda i: (i, 0)),
            pl.BlockSpec((tb, D), lambda i: (i, 0)),
            pl.BlockSpec((D_OUT, D), lambda i: (0, 0)),
            pl.BlockSpec((D_OUT, D), lambda i: (0, 0)),
        ],
        out_specs=pl.BlockSpec((D_OUT, tb), lambda i: (0, i)),
        out_shape=jax.ShapeDtypeStruct((D_OUT, BP), jnp.float32),
    )(xs, xm, w_self, w_rel)


@jax.jit
def kernel(feat, nodes, neigh_idx, W_self, W_rel):
    nodesp = jnp.pad(nodes, (0, BP - B))
    neigh_t = jnp.pad(neigh_idx, ((0, BP - B), (0, 0))).T.copy()
    xs, xsum = _sc_gather(feat, nodesp, neigh_t)
    out = _tc_combine(xs, xsum, W_self, W_rel)
    return out[:, :B]


# speedup vs baseline: 1.1341x; 1.1341x over previous
import functools

import jax
import jax.numpy as jnp
from jax import lax
from jax.experimental import pallas as pl
from jax.experimental.pallas import tpu as pltpu
from jax.experimental.pallas import tpu_sc as plsc

N_NODES = 100000
D = 128
D_OUT = 128
B = 20000
K = 25

NC = 2
NS = 16
NW = NC * NS
BP = 20480
C = 64
NCH_FAST = 17
NCH_SLOW = 3
FAST_CORE = 0
W_FAST = NCH_FAST * C
W_SLOW = NCH_SLOW * C
SPLIT = NS * W_FAST

NBUF = 4
NPAIR = K // 2


def _sc_body(feat, nodesp, neigh_t, xs_out, xsum_out,
             idxv, sidx, planes, selfbuf, acc,
             sem_s, sem_p0, sem_p1, sem_p2, sem_p3):
    c_idx = lax.axis_index("c")
    s_idx = lax.axis_index("s")
    psems = [sem_p0, sem_p1, sem_p2, sem_p3]

    def chunk_at(base):
        pltpu.sync_copy(neigh_t.at[pl.ds(base * K, K * C)], idxv)
        pltpu.sync_copy(nodesp.at[pl.ds(base, C)], sidx)
        cp_self = pltpu.async_copy(feat.at[sidx], selfbuf, sem_s)
        cps = [None] * NBUF
        for b in range(NBUF):
            cps[b] = pltpu.async_copy(feat.at[idxv.at[pl.ds(b * C, C)]],
                                      planes.at[b], psems[b])
        for kp in range(NPAIR):
            ba = (2 * kp) % NBUF
            bb = (2 * kp + 1) % NBUF
            cps[ba].wait()
            cps[bb].wait()
            first = kp == 0

            @plsc.parallel_loop(0, C, unroll=2)
            def _(c, _ba=ba, _bb=bb, _first=first):
                for j in range(8):
                    sl = pl.ds(j * 16, 16)
                    v = planes[_ba, c, sl] + planes[_bb, c, sl]
                    if _first:
                        acc[c, sl] = v
                    else:
                        plsc.addupdate(acc.at[c, sl], v)

            for k in (2 * kp + NBUF, 2 * kp + NBUF + 1):
                if k < K:
                    cps[k % NBUF] = pltpu.async_copy(
                        feat.at[idxv.at[pl.ds(k * C, C)]],
                        planes.at[k % NBUF], psems[k % NBUF])
        cps[(K - 1) % NBUF].wait()

        @plsc.parallel_loop(0, C, unroll=2)
        def _(c):
            for j in range(8):
                sl = pl.ds(j * 16, 16)
                plsc.addupdate(acc.at[c, sl], planes[(K - 1) % NBUF, c, sl])

        cp_self.wait()
        pltpu.sync_copy(acc, xsum_out.at[pl.ds(base, C)])
        pltpu.sync_copy(selfbuf, xs_out.at[pl.ds(base, C)])

    @pl.when(c_idx == FAST_CORE)
    def _():
        def chunk(i, carry):
            chunk_at(s_idx * W_FAST + i * C)
            return carry
        lax.fori_loop(0, NCH_FAST, chunk, 0)

    @pl.when(c_idx != FAST_CORE)
    def _():
        def chunk(i, carry):
            chunk_at(SPLIT + s_idx * W_SLOW + i * C)
            return carry
        lax.fori_loop(0, NCH_SLOW, chunk, 0)


def _sc_gather(feat, nodesp, neigh_t):
    mesh = plsc.VectorSubcoreMesh(core_axis_name="c", subcore_axis_name="s")
    f = pl.kernel(
        _sc_body, mesh=mesh,
        out_type=(jax.ShapeDtypeStruct((BP, D), jnp.float32),
                  jax.ShapeDtypeStruct((BP, D), jnp.float32)),
        scratch_types=[
            pltpu.VMEM((K * C,), jnp.int32),
            pltpu.VMEM((C,), jnp.int32),
            pltpu.VMEM((NBUF, C, D), jnp.float32),
            pltpu.VMEM((C, D), jnp.float32),
            pltpu.VMEM((C, D), jnp.float32),
            pltpu.SemaphoreType.DMA,
            pltpu.SemaphoreType.DMA,
            pltpu.SemaphoreType.DMA,
            pltpu.SemaphoreType.DMA,
            pltpu.SemaphoreType.DMA,
        ],
    )
    return f(feat, nodesp, neigh_t)


def _mm_body(xs_ref, xm_ref, ws_ref, wr_ref, o_ref):
    a = lax.dot_general(ws_ref[...], xs_ref[...],
                        (((1,), (1,)), ((), ())),
                        preferred_element_type=jnp.float32)
    b = lax.dot_general(wr_ref[...], xm_ref[...],
                        (((1,), (1,)), ((), ())),
                        preferred_element_type=jnp.float32)
    o_ref[...] = jnp.maximum(0.5 * a + (0.5 / K) * b, 0.0)


def _tc_combine(xs, xm, w_self, w_rel):
    tb = 2560
    grid = BP // tb
    return pl.pallas_call(
        _mm_body,
        grid=(grid,),
        in_specs=[
            pl.BlockSpec((tb, D), lambda i: (i, 0)),
            pl.BlockSpec((tb, D), lambda i: (i, 0)),
            pl.BlockSpec((D_OUT, D), lambda i: (0, 0)),
            pl.BlockSpec((D_OUT, D), lambda i: (0, 0)),
        ],
        out_specs=pl.BlockSpec((D_OUT, tb), lambda i: (0, i)),
        out_shape=jax.ShapeDtypeStruct((D_OUT, BP), jnp.float32),
    )(xs, xm, w_self, w_rel)


@jax.jit
def kernel(feat, nodes, neigh_idx, W_self, W_rel):
    nodesp = jnp.pad(nodes, (0, BP - B))
    neigh_t = (jnp.pad(neigh_idx, ((0, BP - B), (0, 0)))
               .reshape(BP // C, C, K).transpose(0, 2, 1).reshape(-1))
    xs, xsum = _sc_gather(feat, nodesp, neigh_t)
    out = _tc_combine(xs, xsum, W_self, W_rel)
    return out[:, :B]


# speedup vs baseline: 1.1822x; 1.0424x over previous
import functools

import jax
import jax.numpy as jnp
from jax import lax
from jax.experimental import pallas as pl
from jax.experimental.pallas import tpu as pltpu
from jax.experimental.pallas import tpu_sc as plsc

N_NODES = 100000
D = 128
D_OUT = 128
B = 20000
K = 25

NC = 2
NS = 16
NW = NC * NS
BP = 20480
C = 64
NCH_FAST = 17
NCH_SLOW = 3
FAST_CORE = 1
W_FAST = NCH_FAST * C
W_SLOW = NCH_SLOW * C
SPLIT = NS * W_FAST

NBUF = 4
NPAIR = K // 2


def _sc_body(feat, nodesp, neigh_t, xs_out, xsum_out,
             idxv, sidx, planes, selfbuf, acc,
             sem_s, sem_p0, sem_p1, sem_p2, sem_p3):
    c_idx = lax.axis_index("c")
    s_idx = lax.axis_index("s")
    psems = [sem_p0, sem_p1, sem_p2, sem_p3]

    def chunk_at(base):
        pltpu.sync_copy(neigh_t.at[pl.ds(base * K, K * C)], idxv)
        pltpu.sync_copy(nodesp.at[pl.ds(base, C)], sidx)
        cp_self = pltpu.async_copy(feat.at[sidx], selfbuf, sem_s)
        cps = [None] * NBUF
        for b in range(NBUF):
            cps[b] = pltpu.async_copy(feat.at[idxv.at[pl.ds(b * C, C)]],
                                      planes.at[b], psems[b])
        for kp in range(NPAIR):
            ba = (2 * kp) % NBUF
            bb = (2 * kp + 1) % NBUF
            cps[ba].wait()
            cps[bb].wait()
            first = kp == 0

            @plsc.parallel_loop(0, C, unroll=2)
            def _(c, _ba=ba, _bb=bb, _first=first):
                for j in range(8):
                    sl = pl.ds(j * 16, 16)
                    v = planes[_ba, c, sl] + planes[_bb, c, sl]
                    if _first:
                        acc[c, sl] = v
                    else:
                        plsc.addupdate(acc.at[c, sl], v)

            for k in (2 * kp + NBUF, 2 * kp + NBUF + 1):
                if k < K:
                    cps[k % NBUF] = pltpu.async_copy(
                        feat.at[idxv.at[pl.ds(k * C, C)]],
                        planes.at[k % NBUF], psems[k % NBUF])
        cps[(K - 1) % NBUF].wait()

        @plsc.parallel_loop(0, C, unroll=2)
        def _(c):
            for j in range(8):
                sl = pl.ds(j * 16, 16)
                plsc.addupdate(acc.at[c, sl], planes[(K - 1) % NBUF, c, sl])

        cp_self.wait()
        pltpu.sync_copy(acc, xsum_out.at[pl.ds(base, C)])
        pltpu.sync_copy(selfbuf, xs_out.at[pl.ds(base, C)])

    @pl.when(c_idx == FAST_CORE)
    def _():
        def chunk(i, carry):
            chunk_at(s_idx * W_FAST + i * C)
            return carry
        lax.fori_loop(0, NCH_FAST, chunk, 0)

    @pl.when(c_idx != FAST_CORE)
    def _():
        def chunk(i, carry):
            chunk_at(SPLIT + s_idx * W_SLOW + i * C)
            return carry
        lax.fori_loop(0, NCH_SLOW, chunk, 0)


def _sc_gather(feat, nodesp, neigh_t):
    mesh = plsc.VectorSubcoreMesh(core_axis_name="c", subcore_axis_name="s")
    f = pl.kernel(
        _sc_body, mesh=mesh,
        out_type=(jax.ShapeDtypeStruct((BP, D), jnp.float32),
                  jax.ShapeDtypeStruct((BP, D), jnp.float32)),
        scratch_types=[
            pltpu.VMEM((K * C,), jnp.int32),
            pltpu.VMEM((C,), jnp.int32),
            pltpu.VMEM((NBUF, C, D), jnp.float32),
            pltpu.VMEM((C, D), jnp.float32),
            pltpu.VMEM((C, D), jnp.float32),
            pltpu.SemaphoreType.DMA,
            pltpu.SemaphoreType.DMA,
            pltpu.SemaphoreType.DMA,
            pltpu.SemaphoreType.DMA,
            pltpu.SemaphoreType.DMA,
        ],
    )
    return f(feat, nodesp, neigh_t)


def _mm_body(xs_ref, xm_ref, ws_ref, wr_ref, o_ref):
    a = lax.dot_general(ws_ref[...], xs_ref[...],
                        (((1,), (1,)), ((), ())),
                        preferred_element_type=jnp.float32)
    b = lax.dot_general(wr_ref[...], xm_ref[...],
                        (((1,), (1,)), ((), ())),
                        preferred_element_type=jnp.float32)
    o_ref[...] = jnp.maximum(0.5 * a + (0.5 / K) * b, 0.0)


def _tc_combine(xs, xm, w_self, w_rel):
    tb = 2560
    grid = BP // tb
    return pl.pallas_call(
        _mm_body,
        grid=(grid,),
        in_specs=[
            pl.BlockSpec((tb, D), lambda i: (i, 0)),
            pl.BlockSpec((tb, D), lambda i: (i, 0)),
            pl.BlockSpec((D_OUT, D), lambda i: (0, 0)),
            pl.BlockSpec((D_OUT, D), lambda i: (0, 0)),
        ],
        out_specs=pl.BlockSpec((D_OUT, tb), lambda i: (0, i)),
        out_shape=jax.ShapeDtypeStruct((D_OUT, BP), jnp.float32),
    )(xs, xm, w_self, w_rel)


@jax.jit
def kernel(feat, nodes, neigh_idx, W_self, W_rel):
    nodesp = jnp.pad(nodes, (0, BP - B))
    neigh_t = (jnp.pad(neigh_idx, ((0, BP - B), (0, 0)))
               .reshape(BP // C, C, K).transpose(0, 2, 1).reshape(-1))
    xs, xsum = _sc_gather(feat, nodesp, neigh_t)
    out = _tc_combine(xs, xsum, W_self, W_rel)
    return out[:, :B]


# speedup vs baseline: 3.9859x; 3.3715x over previous
import functools

import jax
import jax.numpy as jnp
from jax import lax
from jax.experimental import pallas as pl
from jax.experimental.pallas import tpu as pltpu
from jax.experimental.pallas import tpu_sc as plsc

N_NODES = 100000
D = 128
D_OUT = 128
B = 20000
K = 25

NC = 2
NS = 16
NW = NC * NS
BP = 20480
C = 64
NCH_FAST = 10
NCH_SLOW = 10
FAST_CORE = 0
W_FAST = NCH_FAST * C
W_SLOW = NCH_SLOW * C
SPLIT = NS * W_FAST

NBUF = 4
NPAIR = K // 2


def _sc_body(feat, nodesp, neigh_t, xs_out, xsum_out,
             idxv, sidx, planes, selfbuf, acc,
             sem_s, sem_p0, sem_p1, sem_p2, sem_p3):
    c_idx = lax.axis_index("c")
    s_idx = lax.axis_index("s")
    psems = [sem_p0, sem_p1, sem_p2, sem_p3]

    def chunk_at(base):
        pltpu.sync_copy(neigh_t.at[pl.ds(base * K, K * C)], idxv)
        pltpu.sync_copy(nodesp.at[pl.ds(base, C)], sidx)
        cp_self = pltpu.async_copy(feat.at[sidx], selfbuf, sem_s)
        cps = [None] * NBUF
        for b in range(NBUF):
            cps[b] = pltpu.async_copy(feat.at[idxv.at[pl.ds(b * C, C)]],
                                      planes.at[b], psems[b])
        for kp in range(NPAIR):
            ba = (2 * kp) % NBUF
            bb = (2 * kp + 1) % NBUF
            cps[ba].wait()
            cps[bb].wait()
            first = kp == 0

            @plsc.parallel_loop(0, C, unroll=2)
            def _(c, _ba=ba, _bb=bb, _first=first):
                for j in range(8):
                    sl = pl.ds(j * 16, 16)
                    v = planes[_ba, c, sl] + planes[_bb, c, sl]
                    if _first:
                        acc[c, sl] = v
                    else:
                        plsc.addupdate(acc.at[c, sl], v)

            for k in (2 * kp + NBUF, 2 * kp + NBUF + 1):
                if k < K:
                    cps[k % NBUF] = pltpu.async_copy(
                        feat.at[idxv.at[pl.ds(k * C, C)]],
                        planes.at[k % NBUF], psems[k % NBUF])
        cps[(K - 1) % NBUF].wait()

        @plsc.parallel_loop(0, C, unroll=2)
        def _(c):
            for j in range(8):
                sl = pl.ds(j * 16, 16)
                plsc.addupdate(acc.at[c, sl], planes[(K - 1) % NBUF, c, sl])

        cp_self.wait()
        pltpu.sync_copy(acc, xsum_out.at[pl.ds(base, C)])
        pltpu.sync_copy(selfbuf, xs_out.at[pl.ds(base, C)])

    @pl.when(c_idx == FAST_CORE)
    def _():
        def chunk(i, carry):
            chunk_at(s_idx * W_FAST + i * C)
            return carry
        lax.fori_loop(0, NCH_FAST, chunk, 0)

    @pl.when(c_idx != FAST_CORE)
    def _():
        def chunk(i, carry):
            chunk_at(SPLIT + s_idx * W_SLOW + i * C)
            return carry
        lax.fori_loop(0, NCH_SLOW, chunk, 0)


def _sc_gather(feat, nodesp, neigh_t):
    mesh = plsc.VectorSubcoreMesh(core_axis_name="c", subcore_axis_name="s")
    f = pl.kernel(
        _sc_body, mesh=mesh,
        out_type=(jax.ShapeDtypeStruct((BP, D), jnp.float32),
                  jax.ShapeDtypeStruct((BP, D), jnp.float32)),
        scratch_types=[
            pltpu.VMEM((K * C,), jnp.int32),
            pltpu.VMEM((C,), jnp.int32),
            pltpu.VMEM((NBUF, C, D), jnp.float32),
            pltpu.VMEM((C, D), jnp.float32),
            pltpu.VMEM((C, D), jnp.float32),
            pltpu.SemaphoreType.DMA,
            pltpu.SemaphoreType.DMA,
            pltpu.SemaphoreType.DMA,
            pltpu.SemaphoreType.DMA,
            pltpu.SemaphoreType.DMA,
        ],
    )
    return f(feat, nodesp, neigh_t)


def _mm_body(xs_ref, xm_ref, ws_ref, wr_ref, o_ref):
    a = lax.dot_general(ws_ref[...], xs_ref[...],
                        (((1,), (1,)), ((), ())),
                        preferred_element_type=jnp.float32)
    b = lax.dot_general(wr_ref[...], xm_ref[...],
                        (((1,), (1,)), ((), ())),
                        preferred_element_type=jnp.float32)
    o_ref[...] = jnp.maximum(0.5 * a + (0.5 / K) * b, 0.0)


def _tc_combine(xs, xm, w_self, w_rel):
    tb = 2560
    grid = BP // tb
    return pl.pallas_call(
        _mm_body,
        grid=(grid,),
        in_specs=[
            pl.BlockSpec((tb, D), lambda i: (i, 0)),
            pl.BlockSpec((tb, D), lambda i: (i, 0)),
            pl.BlockSpec((D_OUT, D), lambda i: (0, 0)),
            pl.BlockSpec((D_OUT, D), lambda i: (0, 0)),
        ],
        out_specs=pl.BlockSpec((D_OUT, tb), lambda i: (0, i)),
        out_shape=jax.ShapeDtypeStruct((D_OUT, BP), jnp.float32),
    )(xs, xm, w_self, w_rel)


@jax.jit
def kernel(feat, nodes, neigh_idx, W_self, W_rel):
    pad_n = jnp.arange(B, BP, dtype=jnp.int32) % N_NODES
    nodesp = jnp.concatenate([nodes, pad_n])
    pad_block = (jnp.arange((BP - B) * K, dtype=jnp.int32)
                 % N_NODES).reshape(BP - B, K)
    neigh_t = (jnp.concatenate([neigh_idx, pad_block], axis=0)
               .reshape(BP // C, C, K).transpose(0, 2, 1).reshape(-1))
    xs, xsum = _sc_gather(feat, nodesp, neigh_t)
    out = _tc_combine(xs, xsum, W_self, W_rel)
    return out[:, :B]


# speedup vs baseline: 4.6614x; 1.1695x over previous
import functools

import numpy as np
import jax
import jax.numpy as jnp
from jax import lax
from jax.experimental import pallas as pl
from jax.experimental.pallas import tpu as pltpu
from jax.experimental.pallas import tpu_sc as plsc

N_NODES = 100000
D = 128
D_OUT = 128
B = 20000
K = 25

NC = 2
NS = 16
NW = NC * NS
BP = 20480
C = 64
NCH = BP // (NW * C)
W_SUB = NCH * C

NBUF = 8
NPAIR = K // 2
KC = K * C


def _sc_body(feat, nodesp, neigh_t, xs_out, xsum_out,
             idxall, sidxall, planes, selfbuf, acc,
             sem_s, sem_p0, sem_p1, sem_p2, sem_p3,
             sem_p4, sem_p5, sem_p6, sem_p7):
    c_idx = lax.axis_index("c")
    s_idx = lax.axis_index("s")
    psems = [sem_p0, sem_p1, sem_p2, sem_p3, sem_p4, sem_p5, sem_p6, sem_p7]
    wid = s_idx * NC + c_idx
    start = wid * W_SUB

    pltpu.sync_copy(neigh_t.at[pl.ds(start * K, NCH * KC)], idxall)
    pltpu.sync_copy(nodesp.at[pl.ds(start, NCH * C)], sidxall)

    def chunk(i, carry):
        base = start + i * C
        ib = i * KC
        cp_self = pltpu.async_copy(
            feat.at[sidxall.at[pl.ds(i * C, C)]], selfbuf, sem_s)
        cps = [None] * NBUF
        for b in range(NBUF):
            cps[b] = pltpu.async_copy(
                feat.at[idxall.at[pl.ds(ib + b * C, C)]],
                planes.at[b], psems[b])
        for kp in range(NPAIR):
            ba = (2 * kp) % NBUF
            bb = (2 * kp + 1) % NBUF
            cps[ba].wait()
            cps[bb].wait()
            first = kp == 0

            @plsc.parallel_loop(0, C, unroll=2)
            def _(c, _ba=ba, _bb=bb, _first=first):
                for j in range(8):
                    sl = pl.ds(j * 16, 16)
                    v = planes[_ba, c, sl] + planes[_bb, c, sl]
                    if _first:
                        acc[c, sl] = v
                    else:
                        plsc.addupdate(acc.at[c, sl], v)

            for k in (2 * kp + NBUF, 2 * kp + NBUF + 1):
                if k < K:
                    cps[k % NBUF] = pltpu.async_copy(
                        feat.at[idxall.at[pl.ds(ib + k * C, C)]],
                        planes.at[k % NBUF], psems[k % NBUF])
        cps[(K - 1) % NBUF].wait()

        @plsc.parallel_loop(0, C, unroll=2)
        def _(c):
            for j in range(8):
                sl = pl.ds(j * 16, 16)
                plsc.addupdate(acc.at[c, sl], planes[(K - 1) % NBUF, c, sl])

        cp_self.wait()
        pltpu.sync_copy(acc, xsum_out.at[pl.ds(base, C)])
        pltpu.sync_copy(selfbuf, xs_out.at[pl.ds(base, C)])
        return carry

    lax.fori_loop(0, NCH, chunk, 0)


def _sc_gather(feat, nodesp, neigh_t):
    mesh = plsc.VectorSubcoreMesh(core_axis_name="c", subcore_axis_name="s")
    f = pl.kernel(
        _sc_body, mesh=mesh,
        out_type=(jax.ShapeDtypeStruct((BP, D), jnp.float32),
                  jax.ShapeDtypeStruct((BP, D), jnp.float32)),
        scratch_types=[
            pltpu.VMEM((NCH * KC,), jnp.int32),
            pltpu.VMEM((NCH * C,), jnp.int32),
            pltpu.VMEM((NBUF, C, D), jnp.float32),
            pltpu.VMEM((C, D), jnp.float32),
            pltpu.VMEM((C, D), jnp.float32),
        ] + [pltpu.SemaphoreType.DMA] * 9,
    )
    return f(feat, nodesp, neigh_t)


def _mm_body(xs_ref, xm_ref, ws_ref, wr_ref, o_ref):
    a = lax.dot_general(ws_ref[...], xs_ref[...],
                        (((1,), (1,)), ((), ())),
                        preferred_element_type=jnp.float32)
    b = lax.dot_general(wr_ref[...], xm_ref[...],
                        (((1,), (1,)), ((), ())),
                        preferred_element_type=jnp.float32)
    o_ref[...] = jnp.maximum(0.5 * a + (0.5 / K) * b, 0.0)


def _tc_combine(xs, xm, w_self, w_rel):
    return pl.pallas_call(
        _mm_body,
        grid=(1,),
        in_specs=[
            pl.BlockSpec((B, D), lambda i: (0, 0)),
            pl.BlockSpec((B, D), lambda i: (0, 0)),
            pl.BlockSpec((D_OUT, D), lambda i: (0, 0)),
            pl.BlockSpec((D_OUT, D), lambda i: (0, 0)),
        ],
        out_specs=pl.BlockSpec((D_OUT, B), lambda i: (0, 0)),
        out_shape=jax.ShapeDtypeStruct((D_OUT, B), jnp.float32),
    )(xs, xm, w_self, w_rel)


@jax.jit
def kernel(feat, nodes, neigh_idx, W_self, W_rel):
    pad_n = jnp.arange(B, BP, dtype=jnp.int32) % N_NODES
    nodesp = jnp.concatenate([nodes, pad_n])
    pad_block = (jnp.arange((BP - B) * K, dtype=jnp.int32)
                 % N_NODES).reshape(BP - B, K)
    neigh_t = (jnp.concatenate([neigh_idx, pad_block], axis=0)
               .reshape(BP // C, C, K).transpose(0, 2, 1).reshape(-1))
    xs, xsum = _sc_gather(feat, nodesp, neigh_t)
    return _tc_combine(xs, xsum, W_self, W_rel)
